# baseline (speedup 1.0000x reference)
import jax
import jax.numpy as jnp
from jax.experimental import pallas as pl
from jax.experimental.pallas import tpu as pltpu

BM = 200
NBUF = 4
DEPTH = 3


def _chunk_copy(adj_ref, buf_ref, sem_ref, c):
    plane = c // (adj_ref.shape[1] // BM)
    i = c % (adj_ref.shape[1] // BM)
    return pltpu.make_async_copy(
        adj_ref.at[plane, pl.ds(i * BM, BM), :],
        buf_ref.at[c % NBUF],
        sem_ref.at[c % NBUF],
    )


def _gcn_kernel(adj_ref, x_ref, w1_ref, b1_ref, w2_ref, b2_ref,
                o_ref, buf_ref, s1_ref, s2_ref, sem_ref):
    n = adj_ref.shape[1]
    nb = n // BM
    nchunks = 2 * nb

    for c in range(DEPTH):
        _chunk_copy(adj_ref, buf_ref, sem_ref, c).start()

    s1_ref[:] = jnp.dot(x_ref[:], w1_ref[:], preferred_element_type=jnp.float32)

    def body(c, _):
        _chunk_copy(adj_ref, buf_ref, sem_ref, c).wait()

        @pl.when(c + DEPTH < nchunks)
        def _prefetch():
            _chunk_copy(adj_ref, buf_ref, sem_ref, c + DEPTH).start()

        row0 = (c % nb) * BM
        a = buf_ref[c % NBUF]

        @pl.when(c < nb)
        def _layer1():
            h = jnp.maximum(
                jnp.dot(a, s1_ref[:], preferred_element_type=jnp.float32)
                + b1_ref[:], 0.0)
            s2_ref[pl.ds(row0, BM), :] = jnp.dot(
                h, w2_ref[:], preferred_element_type=jnp.float32)

        @pl.when(c >= nb)
        def _layer2():
            g = jnp.dot(a, s2_ref[:], preferred_element_type=jnp.float32) \
                + b2_ref[:]
            m = jnp.max(g, axis=1, keepdims=True)
            sh = g - m
            lse = jnp.log(jnp.sum(jnp.exp(sh), axis=1, keepdims=True))
            o_ref[pl.ds(row0, BM), :] = sh - lse

        return 0

    jax.lax.fori_loop(0, nchunks, body, 0)


@jax.jit
def kernel(x, adj, W1, b1, W2, b2):
    n = adj.shape[1]
    nhid = W1.shape[1]
    nclass = W2.shape[1]
    return pl.pallas_call(
        _gcn_kernel,
        in_specs=[
            pl.BlockSpec(memory_space=pltpu.MemorySpace.HBM),
            pl.BlockSpec(x.shape, lambda: (0, 0)),
            pl.BlockSpec(W1.shape, lambda: (0, 0)),
            pl.BlockSpec((1, nhid), lambda: (0, 0)),
            pl.BlockSpec(W2.shape, lambda: (0, 0)),
            pl.BlockSpec((1, nclass), lambda: (0, 0)),
        ],
        out_specs=pl.BlockSpec((n, nclass), lambda: (0, 0)),
        out_shape=jax.ShapeDtypeStruct((n, nclass), jnp.float32),
        scratch_shapes=[
            pltpu.VMEM((NBUF, BM, n), jnp.float32),
            pltpu.VMEM((n, nhid), jnp.float32),
            pltpu.VMEM((n, nclass), jnp.float32),
            pltpu.SemaphoreType.DMA((NBUF,)),
        ],
    )(adj, x, W1, b1.reshape(1, -1), W2, b2.reshape(1, -1))


# speedup vs baseline: 1.0025x; 1.0025x over previous
import jax
import jax.numpy as jnp
from jax.experimental import pallas as pl
from jax.experimental.pallas import tpu as pltpu

BM = 200
NBUF = 4
DEPTH = 3


def _chunk_copy(adj_ref, buf_ref, sem_ref, c, slot):
    nb = adj_ref.shape[1] // BM
    plane = c // nb
    i = c % nb
    return pltpu.make_async_copy(
        adj_ref.at[plane, pl.ds(i * BM, BM), :],
        buf_ref.at[slot],
        sem_ref.at[slot],
    )


def _gcn_kernel(adj_ref, x_ref, w1_ref, b1_ref, w2_ref, b2_ref,
                o_ref, buf_ref, s1_ref, s2_ref, sem_ref):
    n = adj_ref.shape[1]
    nb = n // BM
    nchunks = 2 * nb

    for k in range(DEPTH):
        _chunk_copy(adj_ref, buf_ref, sem_ref, k, k).start()

    s1_ref[:] = jnp.dot(x_ref[:], w1_ref[:], preferred_element_type=jnp.float32)

    def group(g, _):
        for k in range(NBUF):
            c = g * NBUF + k
            _chunk_copy(adj_ref, buf_ref, sem_ref, c, k).wait()

            @pl.when(c + DEPTH < nchunks)
            def _prefetch():
                _chunk_copy(adj_ref, buf_ref, sem_ref, c + DEPTH,
                            (k + DEPTH) % NBUF).start()

            row0 = (c % nb) * BM
            a = buf_ref.at[k]

            @pl.when(c < nb)
            def _layer1():
                h = jnp.maximum(
                    jnp.dot(a[...], s1_ref[:], preferred_element_type=jnp.float32)
                    + b1_ref[:], 0.0)
                s2_ref[pl.ds(row0, BM), :] = jnp.dot(
                    h, w2_ref[:], preferred_element_type=jnp.float32)

            @pl.when(c >= nb)
            def _layer2():
                g2 = jnp.dot(a[...], s2_ref[:], preferred_element_type=jnp.float32) \
                    + b2_ref[:]
                m = jnp.max(g2, axis=1, keepdims=True)
                sh = g2 - m
                lse = jnp.log(jnp.sum(jnp.exp(sh), axis=1, keepdims=True))
                o_ref[pl.ds(row0, BM), :] = sh - lse

        return 0

    jax.lax.fori_loop(0, nchunks // NBUF, group, 0)


@jax.jit
def kernel(x, adj, W1, b1, W2, b2):
    n = adj.shape[1]
    nhid = W1.shape[1]
    nclass = W2.shape[1]
    return pl.pallas_call(
        _gcn_kernel,
        in_specs=[
            pl.BlockSpec(memory_space=pltpu.MemorySpace.HBM),
            pl.BlockSpec(x.shape, lambda: (0, 0)),
            pl.BlockSpec(W1.shape, lambda: (0, 0)),
            pl.BlockSpec((1, nhid), lambda: (0, 0)),
            pl.BlockSpec(W2.shape, lambda: (0, 0)),
            pl.BlockSpec((1, nclass), lambda: (0, 0)),
        ],
        out_specs=pl.BlockSpec((n, nclass), lambda: (0, 0)),
        out_shape=jax.ShapeDtypeStruct((n, nclass), jnp.float32),
        scratch_shapes=[
            pltpu.VMEM((NBUF, BM, n), jnp.float32),
            pltpu.VMEM((n, nhid), jnp.float32),
            pltpu.VMEM((n, nclass), jnp.float32),
            pltpu.SemaphoreType.DMA((NBUF,)),
        ],
    )(adj, x, W1, b1.reshape(1, -1), W2, b2.reshape(1, -1))


# speedup vs baseline: 1.0060x; 1.0035x over previous
import jax
import jax.numpy as jnp
from jax.experimental import pallas as pl
from jax.experimental.pallas import tpu as pltpu

BM = 400


def _gcn_kernel(adj_ref, x_ref, w1_ref, b1_ref, w2_ref, b2_ref,
                o_ref, s1_ref, s2_ref):
    p = pl.program_id(0)
    i = pl.program_id(1)

    @pl.when((p == 0) & (i == 0))
    def _init():
        s1_ref[:] = jnp.dot(x_ref[:], w1_ref[:], preferred_element_type=jnp.float32)

    @pl.when(p == 0)
    def _layer1():
        h = jnp.maximum(
            jnp.dot(adj_ref[0], s1_ref[:], preferred_element_type=jnp.float32)
            + b1_ref[:], 0.0)
        s2 = jnp.dot(h, w2_ref[:], preferred_element_type=jnp.float32)
        s2_ref[pl.ds(i * BM, BM), :] = s2
        o_ref[:] = s2

    @pl.when(p == 1)
    def _layer2():
        g = jnp.dot(adj_ref[0], s2_ref[:], preferred_element_type=jnp.float32) \
            + b2_ref[:]
        m = jnp.max(g, axis=1, keepdims=True)
        sh = g - m
        lse = jnp.log(jnp.sum(jnp.exp(sh), axis=1, keepdims=True))
        o_ref[:] = sh - lse


@jax.jit
def kernel(x, adj, W1, b1, W2, b2):
    n = adj.shape[1]
    nhid = W1.shape[1]
    nclass = W2.shape[1]
    nb = n // BM
    return pl.pallas_call(
        _gcn_kernel,
        grid=(2, nb),
        in_specs=[
            pl.BlockSpec((1, BM, n), lambda p, i: (p, i, 0)),
            pl.BlockSpec(x.shape, lambda p, i: (0, 0)),
            pl.BlockSpec(W1.shape, lambda p, i: (0, 0)),
            pl.BlockSpec((1, nhid), lambda p, i: (0, 0)),
            pl.BlockSpec(W2.shape, lambda p, i: (0, 0)),
            pl.BlockSpec((1, nclass), lambda p, i: (0, 0)),
        ],
        out_specs=pl.BlockSpec((BM, nclass), lambda p, i: (i, 0)),
        out_shape=jax.ShapeDtypeStruct((n, nclass), jnp.float32),
        scratch_shapes=[
            pltpu.VMEM((n, nhid), jnp.float32),
            pltpu.VMEM((n, nclass), jnp.float32),
        ],
        compiler_params=pltpu.CompilerParams(
            dimension_semantics=("arbitrary", "arbitrary"),
        ),
    )(adj, x, W1, b1.reshape(1, -1), W2, b2.reshape(1, -1))


# speedup vs baseline: 1.0079x; 1.0019x over previous
import jax
import jax.numpy as jnp
from jax.experimental import pallas as pl
from jax.experimental.pallas import tpu as pltpu

BM = 400


def _gcn_kernel(adj_ref, x_ref, w1_ref, b1_ref, w2_ref, b2_ref,
                o_ref, s1_ref, s2_ref):
    p = pl.program_id(0)
    i = pl.program_id(1)

    @pl.when((p == 0) & (i == 0))
    def _init():
        s1_ref[:] = jnp.dot(x_ref[:], w1_ref[:], preferred_element_type=jnp.float32)

    @pl.when(p == 0)
    def _layer1():
        h = jnp.maximum(
            jnp.dot(adj_ref[0], s1_ref[:], preferred_element_type=jnp.float32)
            + b1_ref[:], 0.0)
        s2 = jnp.dot(h, w2_ref[:], preferred_element_type=jnp.float32)
        s2_ref[pl.ds(i * BM, BM), :] = s2

    @pl.when(p == 1)
    def _layer2():
        g = jnp.dot(adj_ref[0], s2_ref[:], preferred_element_type=jnp.float32) \
            + b2_ref[:]
        m = jnp.max(g, axis=1, keepdims=True)
        sh = g - m
        lse = jnp.log(jnp.sum(jnp.exp(sh), axis=1, keepdims=True))
        o_ref[:] = sh - lse


@jax.jit
def kernel(x, adj, W1, b1, W2, b2):
    n = adj.shape[1]
    nhid = W1.shape[1]
    nclass = W2.shape[1]
    nb = n // BM
    return pl.pallas_call(
        _gcn_kernel,
        grid=(2, nb),
        in_specs=[
            pl.BlockSpec((1, BM, n), lambda p, i: (p, i, 0)),
            pl.BlockSpec(x.shape, lambda p, i: (0, 0)),
            pl.BlockSpec(W1.shape, lambda p, i: (0, 0)),
            pl.BlockSpec((1, nhid), lambda p, i: (0, 0)),
            pl.BlockSpec(W2.shape, lambda p, i: (0, 0)),
            pl.BlockSpec((1, nclass), lambda p, i: (0, 0)),
        ],
        out_specs=pl.BlockSpec((BM, nclass), lambda p, i: (p * i, 0)),
        out_shape=jax.ShapeDtypeStruct((n, nclass), jnp.float32),
        scratch_shapes=[
            pltpu.VMEM((n, nhid), jnp.float32),
            pltpu.VMEM((n, nclass), jnp.float32),
        ],
        compiler_params=pltpu.CompilerParams(
            dimension_semantics=("arbitrary", "arbitrary"),
        ),
    )(adj, x, W1, b1.reshape(1, -1), W2, b2.reshape(1, -1))
